# Initial kernel scaffold; baseline (speedup 1.0000x reference)
import functools

import jax
import jax.numpy as jnp
from jax import lax
from jax.experimental import pallas as pl
from jax.experimental.pallas import tpu as pltpu
from jax.experimental.pallas import tpu_sc as plsc

_N = 50000
_E = 800000
_D = 64
_G = 8

_BLK = 1024
_NB = 49
_NP = _NB * _BLK
_HALF = _NP // 2
_NC = 2
_NS = 16
_CH = 128
_SCH = 1024
_EPT = 49 * _SCH
_EP = _NS * _EPT
_RPT = _HALF // _NS
_PSCH = 512
_PEPT = _EP // (_NC * _NS)
_PRPT = _NP // _NS

_mesh = plsc.VectorSubcoreMesh(
    core_axis_name="c", subcore_axis_name="s", num_cores=_NC, num_subcores=_NS
)


def _zero_vec_buf(buf, nrows, ncols):
  z = jnp.zeros((16,), jnp.float32)
  for j in range(nrows):
    for k in range(ncols // 16):
      buf[j, k * 16:(k + 1) * 16] = z


def _zero_flat_buf(buf, n):
  z = jnp.zeros((16,), jnp.float32)
  for k in range(n // 16):
    buf[k * 16:(k + 1) * 16] = z


def _prep_body(dst_hbm, wa_hbm, out_hbm, draw, wraw, didx, oebuf, webuf,
               zbuf, accd, accw, sem):
  c = lax.axis_index("c")
  s = lax.axis_index("s")
  wid = c * _NS + s

  _zero_flat_buf(zbuf, 224)
  for k in range(14):
    off = s * _PRPT + k * 224
    pltpu.sync_copy(zbuf, accd.at[pl.ds(off, 224)])
    pltpu.sync_copy(zbuf, accw.at[pl.ds(off, 224)])
  plsc.subcore_barrier()

  nsup = _PEPT // _PSCH

  def _sup(u, carry):
    off = wid * _PEPT + u * _PSCH
    pltpu.sync_copy(dst_hbm.at[pl.ds(off, _PSCH)], draw)
    pltpu.sync_copy(wa_hbm.at[pl.ds(off, _PSCH)], wraw)
    for jc in range(_PSCH // _CH):
      for j in range(8):
        o = jc * _CH + j * 16
        dv = draw[o:o + 16]
        ok = dv >= 0
        didx[jc, j * 16:(j + 1) * 16] = jnp.where(ok, dv, 0)
        oebuf[o:o + 16] = jnp.where(ok, 1.0, 0.0)
        wv = wraw[o:o + 16]
        webuf[o:o + 16] = jnp.where(ok, wv, 0.0)
    for jc in range(_PSCH // _CH):
      pltpu.sync_copy(oebuf.at[pl.ds(jc * _CH, _CH)],
                      accd.at[didx.at[jc]], add=True)
      pltpu.sync_copy(webuf.at[pl.ds(jc * _CH, _CH)],
                      accw.at[didx.at[jc]], add=True)
    return carry

  lax.fori_loop(0, nsup, _sup, None)
  plsc.subcore_barrier()

  for k in range(2):
    off = s * _PRPT + k * _RPT
    pltpu.sync_copy(accd.at[pl.ds(off, _RPT)],
                    out_hbm.at[pl.ds((2 * c + 0) * _NP + off, _RPT)])
    pltpu.sync_copy(accw.at[pl.ds(off, _RPT)],
                    out_hbm.at[pl.ds((2 * c + 1) * _NP + off, _RPT)])


_prep_call = functools.partial(
    pl.kernel,
    out_type=jax.ShapeDtypeStruct((4 * _NP,), jnp.float32),
    mesh=_mesh,
    scratch_types=[
        pltpu.VMEM((_PSCH,), jnp.int32),
        pltpu.VMEM((_PSCH,), jnp.float32),
        pltpu.VMEM((_PSCH // _CH, _CH), jnp.int32),
        pltpu.VMEM((_PSCH,), jnp.float32),
        pltpu.VMEM((_PSCH,), jnp.float32),
        pltpu.VMEM((224,), jnp.float32),
        pltpu.VMEM_SHARED((_NP,), jnp.float32),
        pltpu.VMEM_SHARED((_NP,), jnp.float32),
        pltpu.SemaphoreType.DMA,
    ],
)(_prep_body)


def _edge_body(a_hbm, src_hbm, dst_hbm, w_hbm, out_hbm, sidx, draw, wraw,
               dloc, rows, acc, sem):
  c = lax.axis_index("c")
  s = lax.axis_index("s")
  base = c * _HALF

  _zero_vec_buf(rows, 112, _D)
  for k in range(14):
    pltpu.sync_copy(rows.at[pl.ds(0, 112)],
                    acc.at[pl.ds(s * _RPT + k * 112, 112)])
  plsc.subcore_barrier()

  nsup = _EPT // _SCH
  nchs = _SCH // _CH

  def _sup(u, carry):
    off = s * _EPT + u * _SCH
    pltpu.sync_copy(src_hbm.at[pl.ds(off, _SCH)], sidx)
    pltpu.sync_copy(dst_hbm.at[pl.ds(off, _SCH)], draw)
    pltpu.sync_copy(w_hbm.at[pl.ds(off, _SCH)], wraw)
    for jc in range(nchs):
      for j in range(8):
        o = jc * _CH + j * 16
        dl = draw[o:o + 16] - base
        ok = (dl >= 0) & (dl < _HALF)
        dloc[jc, j * 16:(j + 1) * 16] = jnp.where(ok, dl, 0)
        wv = wraw[o:o + 16]
        wraw[o:o + 16] = jnp.where(ok, wv, 0.0)
    for jc in range(nchs):
      pltpu.async_copy(a_hbm.at[sidx.at[pl.ds(jc * _CH, _CH)]], rows,
                       sem).wait()

      def _scale(e, cc, jc=jc):
        w16 = plsc.load_gather(
            wraw, [jnp.full((16,), jc * _CH + e, jnp.int32)])
        for k in range(4):
          sl = slice(k * 16, (k + 1) * 16)
          rows[e, sl] = rows[e, sl] * w16
        return cc

      lax.fori_loop(0, _CH, _scale, None)
      pltpu.sync_copy(rows, acc.at[dloc.at[jc]], add=True)
    return carry

  lax.fori_loop(0, nsup, _sup, None)
  plsc.subcore_barrier()

  for k in range(7):
    off = s * _RPT + k * 224
    pltpu.sync_copy(acc.at[pl.ds(off, 224)],
                    out_hbm.at[pl.ds(base + off, 224)])


_edge_call = functools.partial(
    pl.kernel,
    out_type=jax.ShapeDtypeStruct((_NP, _D), jnp.float32),
    mesh=_mesh,
    scratch_types=[
        pltpu.VMEM((_SCH,), jnp.int32),
        pltpu.VMEM((_SCH,), jnp.int32),
        pltpu.VMEM((_SCH,), jnp.float32),
        pltpu.VMEM((_SCH // _CH, _CH), jnp.int32),
        pltpu.VMEM((_CH, _D), jnp.float32),
        pltpu.VMEM_SHARED((_HALF, _D), jnp.float32),
        pltpu.SemaphoreType.DMA,
    ],
)(_edge_body)


def _deg_from_ref(dw_ref, col):
  dw = jnp.reshape(dw_ref[...], (4, _BLK))
  return (dw[0 + col] + dw[2 + col])[:, None]


def _pre_math(h, degv, w1, b1, w2, w3, b3):
  a = jnp.dot(h, w1, preferred_element_type=jnp.float32) + b1
  b = jnp.dot(h, w2, preferred_element_type=jnp.float32)
  base = (jnp.dot(h, w3, preferred_element_type=jnp.float32) + b3
          - degv * b)
  return a, base


def _tc_pre_x_body(col, x_ref, dw_ref, w1_ref, b1_ref, w2_ref, w3_ref,
                   b3_ref, a_ref, base_ref):
  degv = _deg_from_ref(dw_ref, col)
  a, base = _pre_math(x_ref[...], degv, w1_ref[...], b1_ref[...],
                      w2_ref[...], w3_ref[...], b3_ref[...])
  a_ref[...] = a
  base_ref[...] = base


def _tc_combine_pre_body(col, relu, bprev_ref, s_ref, dw_ref, w1_ref, b1_ref,
                         w2_ref, w3_ref, b3_ref, a_ref, base_ref):
  h = bprev_ref[...] + s_ref[...]
  if relu:
    h = jnp.maximum(h, 0.0)
  degv = _deg_from_ref(dw_ref, col)
  a, base = _pre_math(h, degv, w1_ref[...], b1_ref[...], w2_ref[...],
                      w3_ref[...], b3_ref[...])
  a_ref[...] = a
  base_ref[...] = base


def _readout_update(h, batch_blk, rmax_ref, rsum_ref, rcnt_ref):
  i = pl.program_id(0)

  @pl.when(i == 0)
  def _init():
    rmax_ref[...] = jnp.full((_G, _D), -jnp.inf, jnp.float32)
    rsum_ref[...] = jnp.zeros((_G, _D), jnp.float32)
    rcnt_ref[...] = jnp.zeros((_G, _D), jnp.float32)

  m = batch_blk[None, :] == jnp.arange(_G, dtype=jnp.int32)[:, None]
  mf = m.astype(jnp.float32)
  rsum_ref[...] += jnp.dot(mf, h, preferred_element_type=jnp.float32)
  rcnt_ref[...] += jnp.sum(mf, axis=1)[:, None]
  upd = jnp.stack([
      jnp.max(jnp.where(m[g][:, None], h, -jnp.inf), axis=0)
      for g in range(_G)
  ])
  rmax_ref[...] = jnp.maximum(rmax_ref[...], upd)


def _tc_combine_read_pre_body(col, bprev_ref, s_ref, dw_ref, batch_ref,
                              w1_ref, b1_ref, w2_ref, w3_ref, b3_ref,
                              a_ref, base_ref, rmax_ref, rsum_ref, rcnt_ref):
  h = bprev_ref[...] + s_ref[...]
  batch_blk = jnp.reshape(batch_ref[...], (_BLK,))
  _readout_update(h, batch_blk, rmax_ref, rsum_ref, rcnt_ref)
  degv = _deg_from_ref(dw_ref, col)
  a, base = _pre_math(h, degv, w1_ref[...], b1_ref[...], w2_ref[...],
                      w3_ref[...], b3_ref[...])
  a_ref[...] = a
  base_ref[...] = base


def _tc_final_body(bprev_ref, s_ref, batch_ref, m1_ref, s1_ref, c1_ref,
                   l1w_ref, l1b_ref, l2w_ref, l2b_ref, out_ref,
                   rmax_ref, rsum_ref, rcnt_ref):
  h = bprev_ref[...] + s_ref[...]
  batch_blk = jnp.reshape(batch_ref[...], (_BLK,))
  _readout_update(h, batch_blk, rmax_ref, rsum_ref, rcnt_ref)

  @pl.when(pl.program_id(0) == _NB - 1)
  def _fin():
    mean1 = s1_ref[...] / jnp.maximum(c1_ref[...], 1.0)
    mean2 = rsum_ref[...] / jnp.maximum(rcnt_ref[...], 1.0)
    z = (jnp.concatenate([m1_ref[...], mean1], axis=1)
         + jnp.concatenate([rmax_ref[...], mean2], axis=1))
    y = jnp.maximum(
        jnp.dot(z, l1w_ref[...], preferred_element_type=jnp.float32)
        + l1b_ref[...], 0.0)
    out_ref[...] = (jnp.sum(y * l2w_ref[...], axis=1, keepdims=True)
                    + l2b_ref[0, 0])


_blk_spec = pl.BlockSpec((_BLK, _D), lambda i: (i, 0))
_dw_spec = pl.BlockSpec((4, 1, 1, _BLK), lambda i: (0, i, 0, 0))
_batch_spec = pl.BlockSpec((1, 1, _BLK), lambda i: (i, 0, 0))


def _full_spec(shape):
  nd = len(shape)
  return pl.BlockSpec(shape, lambda i: (0,) * nd)


_w_spec = _full_spec((_D, _D))
_b_spec = _full_spec((1, _D))
_g_spec = _full_spec((_G, _D))

_nd_out = jax.ShapeDtypeStruct((_NP, _D), jnp.float32)
_g_out = jax.ShapeDtypeStruct((_G, _D), jnp.float32)


def _tc_pre_x(xp, dw4, p):
  return pl.pallas_call(
      functools.partial(_tc_pre_x_body, 0),
      grid=(_NB,),
      in_specs=[_blk_spec, _dw_spec, _w_spec, _b_spec, _w_spec, _w_spec,
                _b_spec],
      out_specs=[_blk_spec, _blk_spec],
      out_shape=[_nd_out, _nd_out],
  )(xp, dw4, p["W1"], p["b1"].reshape(1, _D), p["W2"], p["W3"],
    p["b3"].reshape(1, _D))


def _tc_combine_pre(col, relu, bprev, sarr, dw4, p):
  return pl.pallas_call(
      functools.partial(_tc_combine_pre_body, col, relu),
      grid=(_NB,),
      in_specs=[_blk_spec, _blk_spec, _dw_spec, _w_spec, _b_spec, _w_spec,
                _w_spec, _b_spec],
      out_specs=[_blk_spec, _blk_spec],
      out_shape=[_nd_out, _nd_out],
  )(bprev, sarr, dw4, p["W1"], p["b1"].reshape(1, _D), p["W2"], p["W3"],
    p["b3"].reshape(1, _D))


def _tc_combine_read_pre(col, bprev, sarr, dw4, batch4, p):
  return pl.pallas_call(
      functools.partial(_tc_combine_read_pre_body, col),
      grid=(_NB,),
      in_specs=[_blk_spec, _blk_spec, _dw_spec, _batch_spec, _w_spec,
                _b_spec, _w_spec, _w_spec, _b_spec],
      out_specs=[_blk_spec, _blk_spec, _g_spec, _g_spec, _g_spec],
      out_shape=[_nd_out, _nd_out, _g_out, _g_out, _g_out],
  )(bprev, sarr, dw4, batch4, p["W1"], p["b1"].reshape(1, _D), p["W2"],
    p["W3"], p["b3"].reshape(1, _D))


def _tc_final(bprev, sarr, batch4, m1, s1, c1, params):
  return pl.pallas_call(
      _tc_final_body,
      grid=(_NB,),
      in_specs=[_blk_spec, _blk_spec, _batch_spec, _g_spec, _g_spec, _g_spec,
                _full_spec((2 * _D, _D)), _b_spec, _b_spec,
                _full_spec((1, 1))],
      out_specs=pl.BlockSpec((_G, 1), lambda i: (0, 0)),
      out_shape=jax.ShapeDtypeStruct((_G, 1), jnp.float32),
      scratch_shapes=[
          pltpu.VMEM((_G, _D), jnp.float32),
          pltpu.VMEM((_G, _D), jnp.float32),
          pltpu.VMEM((_G, _D), jnp.float32),
      ],
  )(bprev, sarr, batch4, m1, s1, c1, params["lin1_W"],
    params["lin1_b"].reshape(1, _D), params["lin2_W"].reshape(1, _D),
    params["lin2_b"].reshape(1, 1))


def kernel(x, edge_index, edge_attr, batch, params):
  src = edge_index[0].astype(jnp.int32)
  dst = edge_index[1].astype(jnp.int32)
  pad_e = _EP - _E
  src_p = jnp.concatenate([src, jnp.zeros((pad_e,), jnp.int32)])
  dst_p = jnp.concatenate([dst, jnp.full((pad_e,), -1, jnp.int32)])
  w_one = jnp.concatenate([jnp.ones((_E,), jnp.float32),
                           jnp.zeros((pad_e,), jnp.float32)])
  w_attr = jnp.concatenate([edge_attr.astype(jnp.float32),
                            jnp.zeros((pad_e,), jnp.float32)])
  xp = jnp.pad(x, ((0, _NP - _N), (0, 0)))
  batch4 = jnp.concatenate(
      [batch.astype(jnp.int32), jnp.full((_NP - _N,), 127, jnp.int32)]
  ).reshape(_NB, 1, _BLK)

  dw4 = _prep_call(dst_p, w_attr).reshape(4, _NB, 1, _BLK)

  a1, base1 = _tc_pre_x(xp, dw4, params["conv1"])
  s1 = _edge_call(a1, src_p, dst_p, w_one)
  a2, base2 = _tc_combine_pre(1, True, base1, s1, dw4, params["pool1"])
  s2 = _edge_call(a2, src_p, dst_p, w_attr)
  a3, base3, m1, su1, c1 = _tc_combine_read_pre(
      0, base2, s2, dw4, batch4, params["conv2"])
  s3 = _edge_call(a3, src_p, dst_p, w_one)
  a4, base4 = _tc_combine_pre(1, True, base3, s3, dw4, params["pool2"])
  s4 = _edge_call(a4, src_p, dst_p, w_attr)
  return _tc_final(base4, s4, batch4, m1, su1, c1, params)


# speedup vs baseline: 4.5221x; 4.5221x over previous
import functools

import jax
import jax.numpy as jnp
from jax import lax
from jax.experimental import pallas as pl
from jax.experimental.pallas import tpu as pltpu
from jax.experimental.pallas import tpu_sc as plsc

_N = 50000
_E = 800000
_D = 64
_G = 8

_BLK = 1024
_NB = 49
_NP = _NB * _BLK
_HALF = _NP // 2
_NC = 2
_NS = 16
_CH = 128
_SCH = 1024
_EPT = 49 * _SCH
_EP = _NS * _EPT
_RPT = _HALF // _NS
_PSCH = 512
_PEPT = _EP // (_NC * _NS)
_PRPT = _NP // _NS

_mesh = plsc.VectorSubcoreMesh(
    core_axis_name="c", subcore_axis_name="s", num_cores=_NC, num_subcores=_NS
)


def _zero_vec_buf(buf, nrows, ncols):
  z = jnp.zeros((16,), jnp.float32)
  for j in range(nrows):
    for k in range(ncols // 16):
      buf[j, k * 16:(k + 1) * 16] = z


def _zero_flat_buf(buf, n):
  z = jnp.zeros((16,), jnp.float32)
  for k in range(n // 16):
    buf[k * 16:(k + 1) * 16] = z


def _prep_body(dst_hbm, wa_hbm, out_hbm, draw, wraw, didx, oebuf, webuf,
               zbuf, cbuf, accd, accw, sem):
  c = lax.axis_index("c")
  s = lax.axis_index("s")
  wid = c * _NS + s

  _zero_flat_buf(zbuf, 224)
  for k in range(14):
    off = s * _PRPT + k * 224
    pltpu.sync_copy(zbuf, accd.at[pl.ds(off, 224)])
    pltpu.sync_copy(zbuf, accw.at[pl.ds(off, 224)])
  plsc.subcore_barrier()

  nsup = _PEPT // _PSCH

  def _sup(u, carry):
    off = wid * _PEPT + u * _PSCH
    pltpu.sync_copy(dst_hbm.at[pl.ds(off, _PSCH)], draw)
    pltpu.sync_copy(wa_hbm.at[pl.ds(off, _PSCH)], wraw)
    for jc in range(_PSCH // _CH):
      for j in range(8):
        o = jc * _CH + j * 16
        dv = draw[o:o + 16]
        ok = dv >= 0
        didx[jc, j * 16:(j + 1) * 16] = jnp.where(ok, dv, 0)
        oebuf[o:o + 16] = jnp.where(ok, 1.0, 0.0)
        wv = wraw[o:o + 16]
        webuf[o:o + 16] = jnp.where(ok, wv, 0.0)
    for jc in range(_PSCH // _CH):
      pltpu.sync_copy(oebuf.at[pl.ds(jc * _CH, _CH)],
                      accd.at[didx.at[jc]], add=True)
      pltpu.sync_copy(webuf.at[pl.ds(jc * _CH, _CH)],
                      accw.at[didx.at[jc]], add=True)
    return carry

  lax.fori_loop(0, nsup, _sup, None)
  plsc.subcore_barrier()

  off = s * _PRPT
  pltpu.sync_copy(accd.at[pl.ds(off, _PRPT)], cbuf)
  pltpu.sync_copy(cbuf, out_hbm.at[pl.ds((2 * c + 0) * _NP + off, _PRPT)])
  pltpu.sync_copy(accw.at[pl.ds(off, _PRPT)], cbuf)
  pltpu.sync_copy(cbuf, out_hbm.at[pl.ds((2 * c + 1) * _NP + off, _PRPT)])


_prep_call = functools.partial(
    pl.kernel,
    out_type=jax.ShapeDtypeStruct((4 * _NP,), jnp.float32),
    mesh=_mesh,
    scratch_types=[
        pltpu.VMEM((_PSCH,), jnp.int32),
        pltpu.VMEM((_PSCH,), jnp.float32),
        pltpu.VMEM((_PSCH // _CH, _CH), jnp.int32),
        pltpu.VMEM((_PSCH,), jnp.float32),
        pltpu.VMEM((_PSCH,), jnp.float32),
        pltpu.VMEM((224,), jnp.float32),
        pltpu.VMEM((_PRPT,), jnp.float32),
        pltpu.VMEM_SHARED((_NP,), jnp.float32),
        pltpu.VMEM_SHARED((_NP,), jnp.float32),
        pltpu.SemaphoreType.DMA,
    ],
)(_prep_body)


def _edge_body(a_hbm, src_hbm, dst_hbm, w_hbm, out_hbm, sidx, draw, wraw,
               dloc, rows, acc, sem):
  c = lax.axis_index("c")
  s = lax.axis_index("s")
  base = c * _HALF

  _zero_vec_buf(rows, 112, _D)
  for k in range(14):
    pltpu.sync_copy(rows.at[pl.ds(0, 112)],
                    acc.at[pl.ds(s * _RPT + k * 112, 112)])
  plsc.subcore_barrier()

  nsup = _EPT // _SCH
  nchs = _SCH // _CH

  def _sup(u, carry):
    off = s * _EPT + u * _SCH
    pltpu.sync_copy(src_hbm.at[pl.ds(off, _SCH)], sidx)
    pltpu.sync_copy(dst_hbm.at[pl.ds(off, _SCH)], draw)
    pltpu.sync_copy(w_hbm.at[pl.ds(off, _SCH)], wraw)
    for jc in range(nchs):
      for j in range(8):
        o = jc * _CH + j * 16
        dl = draw[o:o + 16] - base
        ok = (dl >= 0) & (dl < _HALF)
        dloc[jc, j * 16:(j + 1) * 16] = jnp.where(ok, dl, 0)
        wv = wraw[o:o + 16]
        wraw[o:o + 16] = jnp.where(ok, wv, 0.0)
    for jc in range(nchs):
      pltpu.async_copy(a_hbm.at[sidx.at[pl.ds(jc * _CH, _CH)]], rows,
                       sem).wait()

      def _scale(e, cc, jc=jc):
        g = (e // 16) * 16
        w16 = wraw[pl.ds(jc * _CH + g, 16)]
        wsp = w16.at[jnp.full((16,), e - g, jnp.int32)].get(
            mode="promise_in_bounds")
        for k in range(4):
          sl = slice(k * 16, (k + 1) * 16)
          rows[e, sl] = rows[e, sl] * wsp
        return cc

      lax.fori_loop(0, _CH, _scale, None)
      pltpu.sync_copy(rows, acc.at[dloc.at[jc]], add=True)
    return carry

  lax.fori_loop(0, nsup, _sup, None)
  plsc.subcore_barrier()

  for k in range(7):
    off = s * _RPT + k * 224
    pltpu.sync_copy(acc.at[pl.ds(off, 224)],
                    out_hbm.at[pl.ds(base + off, 224)])


_edge_call = functools.partial(
    pl.kernel,
    out_type=jax.ShapeDtypeStruct((_NP, _D), jnp.float32),
    mesh=_mesh,
    compiler_params=pltpu.CompilerParams(use_tc_tiling_on_sc=False),
    scratch_types=[
        pltpu.VMEM((_SCH,), jnp.int32),
        pltpu.VMEM((_SCH,), jnp.int32),
        pltpu.VMEM((_SCH,), jnp.float32),
        pltpu.VMEM((_SCH // _CH, _CH), jnp.int32),
        pltpu.VMEM((_CH, _D), jnp.float32),
        pltpu.VMEM_SHARED((_HALF, _D), jnp.float32),
        pltpu.SemaphoreType.DMA,
    ],
)(_edge_body)


def _deg_from_ref(dw_ref, col):
  dw = jnp.reshape(dw_ref[...], (4, _BLK))
  return (dw[0 + col] + dw[2 + col])[:, None]


def _pre_math(h, degv, w1, b1, w2, w3, b3):
  a = jnp.dot(h, w1, preferred_element_type=jnp.float32) + b1
  b = jnp.dot(h, w2, preferred_element_type=jnp.float32)
  base = (jnp.dot(h, w3, preferred_element_type=jnp.float32) + b3
          - degv * b)
  return a, base


def _tc_pre_x_body(col, x_ref, dw_ref, w1_ref, b1_ref, w2_ref, w3_ref,
                   b3_ref, a_ref, base_ref):
  degv = _deg_from_ref(dw_ref, col)
  a, base = _pre_math(x_ref[...], degv, w1_ref[...], b1_ref[...],
                      w2_ref[...], w3_ref[...], b3_ref[...])
  a_ref[...] = a
  base_ref[...] = base


def _tc_combine_pre_body(col, relu, bprev_ref, s_ref, dw_ref, w1_ref, b1_ref,
                         w2_ref, w3_ref, b3_ref, a_ref, base_ref):
  h = bprev_ref[...] + s_ref[...]
  if relu:
    h = jnp.maximum(h, 0.0)
  degv = _deg_from_ref(dw_ref, col)
  a, base = _pre_math(h, degv, w1_ref[...], b1_ref[...], w2_ref[...],
                      w3_ref[...], b3_ref[...])
  a_ref[...] = a
  base_ref[...] = base


def _readout_update(h, bo, rmax_ref, rsum_ref, rcnt_ref):
  i = pl.program_id(0)

  @pl.when(i == 0)
  def _init():
    rmax_ref[...] = jnp.full((_G, _D), -jnp.inf, jnp.float32)
    rsum_ref[...] = jnp.zeros((_G, _D), jnp.float32)
    rcnt_ref[...] = jnp.zeros((_G, _D), jnp.float32)

  dn = (((0,), (0,)), ((), ()))
  rsum_ref[...] += lax.dot_general(bo, h, dn,
                                   preferred_element_type=jnp.float32)
  rcnt_ref[...] += lax.dot_general(bo, jnp.ones_like(h), dn,
                                   preferred_element_type=jnp.float32)
  parts = [
      jnp.max(jnp.where(bo[:, g:g + 1] > 0.5, h, -jnp.inf), axis=0,
              keepdims=True)
      for g in range(_G)
  ]
  rmax_ref[...] = jnp.maximum(rmax_ref[...], jnp.concatenate(parts, axis=0))


def _tc_combine_read_pre_body(col, bprev_ref, s_ref, dw_ref, bo_ref,
                              w1_ref, b1_ref, w2_ref, w3_ref, b3_ref,
                              a_ref, base_ref, rmax_ref, rsum_ref, rcnt_ref):
  h = bprev_ref[...] + s_ref[...]
  _readout_update(h, bo_ref[...], rmax_ref, rsum_ref, rcnt_ref)
  degv = _deg_from_ref(dw_ref, col)
  a, base = _pre_math(h, degv, w1_ref[...], b1_ref[...], w2_ref[...],
                      w3_ref[...], b3_ref[...])
  a_ref[...] = a
  base_ref[...] = base


def _tc_final_body(bprev_ref, s_ref, bo_ref, m1_ref, s1_ref, c1_ref,
                   l1w_ref, l1b_ref, l2w_ref, l2b_ref, out_ref,
                   rmax_ref, rsum_ref, rcnt_ref):
  h = bprev_ref[...] + s_ref[...]
  _readout_update(h, bo_ref[...], rmax_ref, rsum_ref, rcnt_ref)

  @pl.when(pl.program_id(0) == _NB - 1)
  def _fin():
    mean1 = s1_ref[...] / jnp.maximum(c1_ref[...], 1.0)
    mean2 = rsum_ref[...] / jnp.maximum(rcnt_ref[...], 1.0)
    z = (jnp.concatenate([m1_ref[...], mean1], axis=1)
         + jnp.concatenate([rmax_ref[...], mean2], axis=1))
    y = jnp.maximum(
        jnp.dot(z, l1w_ref[...], preferred_element_type=jnp.float32)
        + l1b_ref[...], 0.0)
    out_ref[...] = (jnp.sum(y * l2w_ref[...], axis=1, keepdims=True)
                    + l2b_ref[0, 0])


_blk_spec = pl.BlockSpec((_BLK, _D), lambda i: (i, 0))
_dw_spec = pl.BlockSpec((4, 1, 1, _BLK), lambda i: (0, i, 0, 0))
_bo_spec = pl.BlockSpec((_BLK, _G), lambda i: (i, 0))


def _full_spec(shape):
  nd = len(shape)
  return pl.BlockSpec(shape, lambda i: (0,) * nd)


_w_spec = _full_spec((_D, _D))
_b_spec = _full_spec((1, _D))
_g_spec = _full_spec((_G, _D))

_nd_out = jax.ShapeDtypeStruct((_NP, _D), jnp.float32)
_g_out = jax.ShapeDtypeStruct((_G, _D), jnp.float32)


def _tc_pre_x(xp, dw4, p):
  return pl.pallas_call(
      functools.partial(_tc_pre_x_body, 0),
      grid=(_NB,),
      in_specs=[_blk_spec, _dw_spec, _w_spec, _b_spec, _w_spec, _w_spec,
                _b_spec],
      out_specs=[_blk_spec, _blk_spec],
      out_shape=[_nd_out, _nd_out],
  )(xp, dw4, p["W1"], p["b1"].reshape(1, _D), p["W2"], p["W3"],
    p["b3"].reshape(1, _D))


def _tc_combine_pre(col, relu, bprev, sarr, dw4, p):
  return pl.pallas_call(
      functools.partial(_tc_combine_pre_body, col, relu),
      grid=(_NB,),
      in_specs=[_blk_spec, _blk_spec, _dw_spec, _w_spec, _b_spec, _w_spec,
                _w_spec, _b_spec],
      out_specs=[_blk_spec, _blk_spec],
      out_shape=[_nd_out, _nd_out],
  )(bprev, sarr, dw4, p["W1"], p["b1"].reshape(1, _D), p["W2"], p["W3"],
    p["b3"].reshape(1, _D))


def _tc_combine_read_pre(col, bprev, sarr, dw4, bo, p):
  return pl.pallas_call(
      functools.partial(_tc_combine_read_pre_body, col),
      grid=(_NB,),
      in_specs=[_blk_spec, _blk_spec, _dw_spec, _bo_spec, _w_spec,
                _b_spec, _w_spec, _w_spec, _b_spec],
      out_specs=[_blk_spec, _blk_spec, _g_spec, _g_spec, _g_spec],
      out_shape=[_nd_out, _nd_out, _g_out, _g_out, _g_out],
  )(bprev, sarr, dw4, bo, p["W1"], p["b1"].reshape(1, _D), p["W2"],
    p["W3"], p["b3"].reshape(1, _D))


def _tc_final(bprev, sarr, bo, m1, s1, c1, params):
  return pl.pallas_call(
      _tc_final_body,
      grid=(_NB,),
      in_specs=[_blk_spec, _blk_spec, _bo_spec, _g_spec, _g_spec, _g_spec,
                _full_spec((2 * _D, _D)), _b_spec, _b_spec,
                _full_spec((1, 1))],
      out_specs=pl.BlockSpec((_G, 1), lambda i: (0, 0)),
      out_shape=jax.ShapeDtypeStruct((_G, 1), jnp.float32),
      scratch_shapes=[
          pltpu.VMEM((_G, _D), jnp.float32),
          pltpu.VMEM((_G, _D), jnp.float32),
          pltpu.VMEM((_G, _D), jnp.float32),
      ],
  )(bprev, sarr, bo, m1, s1, c1, params["lin1_W"],
    params["lin1_b"].reshape(1, _D), params["lin2_W"].reshape(1, _D),
    params["lin2_b"].reshape(1, 1))


def kernel(x, edge_index, edge_attr, batch, params):
  src = edge_index[0].astype(jnp.int32)
  dst = edge_index[1].astype(jnp.int32)
  pad_e = _EP - _E
  src_p = jnp.concatenate([src, jnp.zeros((pad_e,), jnp.int32)])
  dst_p = jnp.concatenate([dst, jnp.full((pad_e,), -1, jnp.int32)])
  w_one = jnp.concatenate([jnp.ones((_E,), jnp.float32),
                           jnp.zeros((pad_e,), jnp.float32)])
  w_attr = jnp.concatenate([edge_attr.astype(jnp.float32),
                            jnp.zeros((pad_e,), jnp.float32)])
  xp = jnp.pad(x, ((0, _NP - _N), (0, 0)))
  batch_p = jnp.concatenate(
      [batch.astype(jnp.int32), jnp.full((_NP - _N,), 127, jnp.int32)])
  bo = (batch_p[:, None] == jnp.arange(_G, dtype=jnp.int32)[None, :]
        ).astype(jnp.float32)

  dw4 = _prep_call(dst_p, w_attr).reshape(4, _NB, 1, _BLK)

  a1, base1 = _tc_pre_x(xp, dw4, params["conv1"])
  s1 = _edge_call(a1, src_p, dst_p, w_one)
  a2, base2 = _tc_combine_pre(1, True, base1, s1, dw4, params["pool1"])
  s2 = _edge_call(a2, src_p, dst_p, w_attr)
  a3, base3, m1, su1, c1 = _tc_combine_read_pre(
      0, base2, s2, dw4, bo, params["conv2"])
  s3 = _edge_call(a3, src_p, dst_p, w_one)
  a4, base4 = _tc_combine_pre(1, True, base3, s3, dw4, params["pool2"])
  s4 = _edge_call(a4, src_p, dst_p, w_attr)
  return _tc_final(base4, s4, bo, m1, su1, c1, params)


# speedup vs baseline: 6.7921x; 1.5020x over previous
import functools

import jax
import jax.numpy as jnp
from jax import lax
from jax.experimental import pallas as pl
from jax.experimental.pallas import tpu as pltpu
from jax.experimental.pallas import tpu_sc as plsc

_N = 50000
_E = 800000
_D = 64
_G = 8

_BLK = 1024
_NB = 49
_NP = _NB * _BLK
_HALF = _NP // 2
_NC = 2
_NS = 16
_CH = 128
_SCH = 1024
_EPT = 49 * _SCH
_EP = _NS * _EPT
_RPT = _HALF // _NS
_PSCH = 512
_PEPT = _EP // (_NC * _NS)
_PRPT = _NP // _NS

_mesh = plsc.VectorSubcoreMesh(
    core_axis_name="c", subcore_axis_name="s", num_cores=_NC, num_subcores=_NS
)


def _zero_vec_buf(buf, nrows, ncols):
  z = jnp.zeros((16,), jnp.float32)
  for j in range(nrows):
    for k in range(ncols // 16):
      buf[j, k * 16:(k + 1) * 16] = z


def _zero_flat_buf(buf, n):
  z = jnp.zeros((16,), jnp.float32)
  for k in range(n // 16):
    buf[k * 16:(k + 1) * 16] = z


def _prep_body(dst_hbm, wa_hbm, out_hbm, draw, wraw, didx, oebuf, webuf,
               zbuf, cbuf, accd, accw, sem):
  c = lax.axis_index("c")
  s = lax.axis_index("s")
  wid = c * _NS + s

  _zero_flat_buf(zbuf, 224)
  for k in range(14):
    off = s * _PRPT + k * 224
    pltpu.sync_copy(zbuf, accd.at[pl.ds(off, 224)])
    pltpu.sync_copy(zbuf, accw.at[pl.ds(off, 224)])
  plsc.subcore_barrier()

  nsup = _PEPT // _PSCH

  def _sup(u, carry):
    off = wid * _PEPT + u * _PSCH
    pltpu.sync_copy(dst_hbm.at[pl.ds(off, _PSCH)], draw)
    pltpu.sync_copy(wa_hbm.at[pl.ds(off, _PSCH)], wraw)
    for jc in range(_PSCH // _CH):
      for j in range(8):
        o = jc * _CH + j * 16
        dv = draw[o:o + 16]
        ok = dv >= 0
        didx[jc, j * 16:(j + 1) * 16] = jnp.where(ok, dv, 0)
        oebuf[o:o + 16] = jnp.where(ok, 1.0, 0.0)
        wv = wraw[o:o + 16]
        webuf[o:o + 16] = jnp.where(ok, wv, 0.0)
    for jc in range(_PSCH // _CH):
      pltpu.sync_copy(oebuf.at[pl.ds(jc * _CH, _CH)],
                      accd.at[didx.at[jc]], add=True)
      pltpu.sync_copy(webuf.at[pl.ds(jc * _CH, _CH)],
                      accw.at[didx.at[jc]], add=True)
    return carry

  lax.fori_loop(0, nsup, _sup, None)
  plsc.subcore_barrier()

  off = s * _PRPT
  pltpu.sync_copy(accd.at[pl.ds(off, _PRPT)], cbuf)
  pltpu.sync_copy(cbuf, out_hbm.at[pl.ds((2 * c + 0) * _NP + off, _PRPT)])
  pltpu.sync_copy(accw.at[pl.ds(off, _PRPT)], cbuf)
  pltpu.sync_copy(cbuf, out_hbm.at[pl.ds((2 * c + 1) * _NP + off, _PRPT)])


_prep_call = functools.partial(
    pl.kernel,
    out_type=jax.ShapeDtypeStruct((4 * _NP,), jnp.float32),
    mesh=_mesh,
    scratch_types=[
        pltpu.VMEM((_PSCH,), jnp.int32),
        pltpu.VMEM((_PSCH,), jnp.float32),
        pltpu.VMEM((_PSCH // _CH, _CH), jnp.int32),
        pltpu.VMEM((_PSCH,), jnp.float32),
        pltpu.VMEM((_PSCH,), jnp.float32),
        pltpu.VMEM((224,), jnp.float32),
        pltpu.VMEM((_PRPT,), jnp.float32),
        pltpu.VMEM_SHARED((_NP,), jnp.float32),
        pltpu.VMEM_SHARED((_NP,), jnp.float32),
        pltpu.SemaphoreType.DMA,
    ],
)(_prep_body)


def _edge_body(has_w, a_hbm, src_hbm, dst_hbm, w_hbm, out_hbm, sidx, draw,
               wraw, dloc, rows0, rows1, acc, gs0, gs1, ss0, ss1):
  c = lax.axis_index("c")
  s = lax.axis_index("s")
  base = c * _HALF
  dump = _HALF + s

  _zero_vec_buf(rows0, 112, _D)
  for k in range(14):
    pltpu.sync_copy(rows0.at[pl.ds(0, 112)],
                    acc.at[pl.ds(s * _RPT + k * 112, 112)])
  plsc.subcore_barrier()

  nsup = _EPT // _SCH
  nchs = _SCH // _CH
  bufs = ((rows0, gs0, ss0), (rows1, gs1, ss1))

  def _sup(u, carry):
    off = s * _EPT + u * _SCH
    pltpu.sync_copy(src_hbm.at[pl.ds(off, _SCH)], sidx)
    pltpu.sync_copy(dst_hbm.at[pl.ds(off, _SCH)], draw)
    if has_w:
      pltpu.sync_copy(w_hbm.at[pl.ds(off, _SCH)], wraw)
    for jc in range(nchs):
      for j in range(8):
        o = jc * _CH + j * 16
        dl = draw[o:o + 16] - base
        ok = (dl >= 0) & (dl < _HALF)
        dloc[jc, j * 16:(j + 1) * 16] = jnp.where(ok, dl, dump)
    gd = [None, None]
    sd = [None, None]
    gd[0] = pltpu.async_copy(a_hbm.at[sidx.at[pl.ds(0, _CH)]], rows0, gs0)
    for jc in range(nchs):
      b = jc % 2
      nb = 1 - b
      rows_b, _, ssem_b = bufs[b]
      gd[b].wait()
      if jc < nchs - 1:
        if jc >= 1:
          sd[nb].wait()
        rows_nb, gsem_nb, _ = bufs[nb]
        gd[nb] = pltpu.async_copy(
            a_hbm.at[sidx.at[pl.ds((jc + 1) * _CH, _CH)]], rows_nb, gsem_nb)
      if has_w:
        def _scale(e, cc, jc=jc, rows=rows_b):
          g = (e // 16) * 16
          w16 = wraw[pl.ds(jc * _CH + g, 16)]
          wsp = w16.at[jnp.full((16,), e - g, jnp.int32)].get(
              mode="promise_in_bounds")
          for k in range(4):
            sl = slice(k * 16, (k + 1) * 16)
            rows[e, sl] = rows[e, sl] * wsp
          return cc

        lax.fori_loop(0, _CH, _scale, None)
      sd[b] = pltpu.async_copy(rows_b, acc.at[dloc.at[jc]], ssem_b,
                               add=True)
    sd[0].wait()
    sd[1].wait()
    return carry

  lax.fori_loop(0, nsup, _sup, None)
  plsc.subcore_barrier()

  for k in range(7):
    off = s * _RPT + k * 224
    pltpu.sync_copy(acc.at[pl.ds(off, 224)],
                    out_hbm.at[pl.ds(base + off, 224)])


def _make_edge_call(has_w):
  return functools.partial(
      pl.kernel,
      out_type=jax.ShapeDtypeStruct((_NP, _D), jnp.float32),
      mesh=_mesh,
      compiler_params=pltpu.CompilerParams(use_tc_tiling_on_sc=False),
      scratch_types=[
          pltpu.VMEM((_SCH,), jnp.int32),
          pltpu.VMEM((_SCH,), jnp.int32),
          pltpu.VMEM((_SCH,), jnp.float32),
          pltpu.VMEM((_SCH // _CH, _CH), jnp.int32),
          pltpu.VMEM((_CH, _D), jnp.float32),
          pltpu.VMEM((_CH, _D), jnp.float32),
          pltpu.VMEM_SHARED((_HALF + _NS, _D), jnp.float32),
          pltpu.SemaphoreType.DMA,
          pltpu.SemaphoreType.DMA,
          pltpu.SemaphoreType.DMA,
          pltpu.SemaphoreType.DMA,
      ],
  )(functools.partial(_edge_body, has_w))


_edge_call_w = _make_edge_call(True)
_edge_call_nw = _make_edge_call(False)


def _deg_from_ref(dw_ref, col):
  dw = jnp.reshape(dw_ref[...], (4, _BLK))
  return (dw[0 + col] + dw[2 + col])[:, None]


def _pre_math(h, degv, w1, b1, w2, w3, b3):
  a = jnp.dot(h, w1, preferred_element_type=jnp.float32) + b1
  b = jnp.dot(h, w2, preferred_element_type=jnp.float32)
  base = (jnp.dot(h, w3, preferred_element_type=jnp.float32) + b3
          - degv * b)
  return a, base


def _tc_pre_x_body(col, x_ref, dw_ref, w1_ref, b1_ref, w2_ref, w3_ref,
                   b3_ref, a_ref, base_ref):
  degv = _deg_from_ref(dw_ref, col)
  a, base = _pre_math(x_ref[...], degv, w1_ref[...], b1_ref[...],
                      w2_ref[...], w3_ref[...], b3_ref[...])
  a_ref[...] = a
  base_ref[...] = base


def _tc_combine_pre_body(col, relu, bprev_ref, s_ref, dw_ref, w1_ref, b1_ref,
                         w2_ref, w3_ref, b3_ref, a_ref, base_ref):
  h = bprev_ref[...] + s_ref[...]
  if relu:
    h = jnp.maximum(h, 0.0)
  degv = _deg_from_ref(dw_ref, col)
  a, base = _pre_math(h, degv, w1_ref[...], b1_ref[...], w2_ref[...],
                      w3_ref[...], b3_ref[...])
  a_ref[...] = a
  base_ref[...] = base


def _readout_update(h, bo, rmax_ref, rsum_ref, rcnt_ref):
  i = pl.program_id(0)

  @pl.when(i == 0)
  def _init():
    rmax_ref[...] = jnp.full((_G, _D), -jnp.inf, jnp.float32)
    rsum_ref[...] = jnp.zeros((_G, _D), jnp.float32)
    rcnt_ref[...] = jnp.zeros((_G, _D), jnp.float32)

  dn = (((0,), (0,)), ((), ()))
  rsum_ref[...] += lax.dot_general(bo, h, dn,
                                   preferred_element_type=jnp.float32)
  rcnt_ref[...] += lax.dot_general(bo, jnp.ones_like(h), dn,
                                   preferred_element_type=jnp.float32)
  parts = [
      jnp.max(jnp.where(bo[:, g:g + 1] > 0.5, h, -jnp.inf), axis=0,
              keepdims=True)
      for g in range(_G)
  ]
  rmax_ref[...] = jnp.maximum(rmax_ref[...], jnp.concatenate(parts, axis=0))


def _tc_combine_read_pre_body(col, bprev_ref, s_ref, dw_ref, bo_ref,
                              w1_ref, b1_ref, w2_ref, w3_ref, b3_ref,
                              a_ref, base_ref, rmax_ref, rsum_ref, rcnt_ref):
  h = bprev_ref[...] + s_ref[...]
  _readout_update(h, bo_ref[...], rmax_ref, rsum_ref, rcnt_ref)
  degv = _deg_from_ref(dw_ref, col)
  a, base = _pre_math(h, degv, w1_ref[...], b1_ref[...], w2_ref[...],
                      w3_ref[...], b3_ref[...])
  a_ref[...] = a
  base_ref[...] = base


def _tc_final_body(bprev_ref, s_ref, bo_ref, m1_ref, s1_ref, c1_ref,
                   l1w_ref, l1b_ref, l2w_ref, l2b_ref, out_ref,
                   rmax_ref, rsum_ref, rcnt_ref):
  h = bprev_ref[...] + s_ref[...]
  _readout_update(h, bo_ref[...], rmax_ref, rsum_ref, rcnt_ref)

  @pl.when(pl.program_id(0) == _NB - 1)
  def _fin():
    mean1 = s1_ref[...] / jnp.maximum(c1_ref[...], 1.0)
    mean2 = rsum_ref[...] / jnp.maximum(rcnt_ref[...], 1.0)
    z = (jnp.concatenate([m1_ref[...], mean1], axis=1)
         + jnp.concatenate([rmax_ref[...], mean2], axis=1))
    y = jnp.maximum(
        jnp.dot(z, l1w_ref[...], preferred_element_type=jnp.float32)
        + l1b_ref[...], 0.0)
    out_ref[...] = (jnp.sum(y * l2w_ref[...], axis=1, keepdims=True)
                    + l2b_ref[0, 0])


_blk_spec = pl.BlockSpec((_BLK, _D), lambda i: (i, 0))
_dw_spec = pl.BlockSpec((4, 1, 1, _BLK), lambda i: (0, i, 0, 0))
_bo_spec = pl.BlockSpec((_BLK, _G), lambda i: (i, 0))


def _full_spec(shape):
  nd = len(shape)
  return pl.BlockSpec(shape, lambda i: (0,) * nd)


_w_spec = _full_spec((_D, _D))
_b_spec = _full_spec((1, _D))
_g_spec = _full_spec((_G, _D))

_nd_out = jax.ShapeDtypeStruct((_NP, _D), jnp.float32)
_g_out = jax.ShapeDtypeStruct((_G, _D), jnp.float32)


def _tc_pre_x(xp, dw4, p):
  return pl.pallas_call(
      functools.partial(_tc_pre_x_body, 0),
      grid=(_NB,),
      in_specs=[_blk_spec, _dw_spec, _w_spec, _b_spec, _w_spec, _w_spec,
                _b_spec],
      out_specs=[_blk_spec, _blk_spec],
      out_shape=[_nd_out, _nd_out],
  )(xp, dw4, p["W1"], p["b1"].reshape(1, _D), p["W2"], p["W3"],
    p["b3"].reshape(1, _D))


def _tc_combine_pre(col, relu, bprev, sarr, dw4, p):
  return pl.pallas_call(
      functools.partial(_tc_combine_pre_body, col, relu),
      grid=(_NB,),
      in_specs=[_blk_spec, _blk_spec, _dw_spec, _w_spec, _b_spec, _w_spec,
                _w_spec, _b_spec],
      out_specs=[_blk_spec, _blk_spec],
      out_shape=[_nd_out, _nd_out],
  )(bprev, sarr, dw4, p["W1"], p["b1"].reshape(1, _D), p["W2"], p["W3"],
    p["b3"].reshape(1, _D))


def _tc_combine_read_pre(col, bprev, sarr, dw4, bo, p):
  return pl.pallas_call(
      functools.partial(_tc_combine_read_pre_body, col),
      grid=(_NB,),
      in_specs=[_blk_spec, _blk_spec, _dw_spec, _bo_spec, _w_spec,
                _b_spec, _w_spec, _w_spec, _b_spec],
      out_specs=[_blk_spec, _blk_spec, _g_spec, _g_spec, _g_spec],
      out_shape=[_nd_out, _nd_out, _g_out, _g_out, _g_out],
  )(bprev, sarr, dw4, bo, p["W1"], p["b1"].reshape(1, _D), p["W2"],
    p["W3"], p["b3"].reshape(1, _D))


def _tc_final(bprev, sarr, bo, m1, s1, c1, params):
  return pl.pallas_call(
      _tc_final_body,
      grid=(_NB,),
      in_specs=[_blk_spec, _blk_spec, _bo_spec, _g_spec, _g_spec, _g_spec,
                _full_spec((2 * _D, _D)), _b_spec, _b_spec,
                _full_spec((1, 1))],
      out_specs=pl.BlockSpec((_G, 1), lambda i: (0, 0)),
      out_shape=jax.ShapeDtypeStruct((_G, 1), jnp.float32),
      scratch_shapes=[
          pltpu.VMEM((_G, _D), jnp.float32),
          pltpu.VMEM((_G, _D), jnp.float32),
          pltpu.VMEM((_G, _D), jnp.float32),
      ],
  )(bprev, sarr, bo, m1, s1, c1, params["lin1_W"],
    params["lin1_b"].reshape(1, _D), params["lin2_W"].reshape(1, _D),
    params["lin2_b"].reshape(1, 1))


def kernel(x, edge_index, edge_attr, batch, params):
  src = edge_index[0].astype(jnp.int32)
  dst = edge_index[1].astype(jnp.int32)
  pad_e = _EP - _E
  src_p = jnp.concatenate([src, jnp.zeros((pad_e,), jnp.int32)])
  dst_p = jnp.concatenate([dst, jnp.full((pad_e,), -1, jnp.int32)])
  w_one = jnp.concatenate([jnp.ones((_E,), jnp.float32),
                           jnp.zeros((pad_e,), jnp.float32)])
  w_attr = jnp.concatenate([edge_attr.astype(jnp.float32),
                            jnp.zeros((pad_e,), jnp.float32)])
  xp = jnp.pad(x, ((0, _NP - _N), (0, 0)))
  batch_p = jnp.concatenate(
      [batch.astype(jnp.int32), jnp.full((_NP - _N,), 127, jnp.int32)])
  bo = (batch_p[:, None] == jnp.arange(_G, dtype=jnp.int32)[None, :]
        ).astype(jnp.float32)

  dw4 = _prep_call(dst_p, w_attr).reshape(4, _NB, 1, _BLK)

  a1, base1 = _tc_pre_x(xp, dw4, params["conv1"])
  s1 = _edge_call_nw(a1, src_p, dst_p, w_one)
  a2, base2 = _tc_combine_pre(1, True, base1, s1, dw4, params["pool1"])
  s2 = _edge_call_w(a2, src_p, dst_p, w_attr)
  a3, base3, m1, su1, c1 = _tc_combine_read_pre(
      0, base2, s2, dw4, bo, params["conv2"])
  s3 = _edge_call_nw(a3, src_p, dst_p, w_one)
  a4, base4 = _tc_combine_pre(1, True, base3, s3, dw4, params["pool2"])
  s4 = _edge_call_w(a4, src_p, dst_p, w_attr)
  return _tc_final(base4, s4, bo, m1, su1, c1, params)


# speedup vs baseline: 7.6055x; 1.1198x over previous
import functools

import jax
import jax.numpy as jnp
from jax import lax
from jax.experimental import pallas as pl
from jax.experimental.pallas import tpu as pltpu
from jax.experimental.pallas import tpu_sc as plsc

_N = 50000
_E = 800000
_D = 64
_G = 8

_BLK = 1024
_NB = 49
_NP = _NB * _BLK
_HALF = _NP // 2
_NC = 2
_NS = 16
_CH = 128
_SCH = 1024
_EPT = 49 * _SCH
_EP = _NS * _EPT
_RPT = _HALF // _NS
_PSCH = 512
_PEPT = _EP // (_NC * _NS)
_PRPT = _NP // _NS

_mesh = plsc.VectorSubcoreMesh(
    core_axis_name="c", subcore_axis_name="s", num_cores=_NC, num_subcores=_NS
)


def _zero_vec_buf(buf, nrows, ncols):
  z = jnp.zeros((16,), jnp.float32)
  for j in range(nrows):
    for k in range(ncols // 16):
      buf[j, k * 16:(k + 1) * 16] = z


def _zero_flat_buf(buf, n):
  z = jnp.zeros((16,), jnp.float32)
  for k in range(n // 16):
    buf[k * 16:(k + 1) * 16] = z


def _prep_body(dst_hbm, wa_hbm, out_hbm, draw, wraw, didx, oebuf, webuf,
               zbuf, cbuf, accd, accw, sem):
  c = lax.axis_index("c")
  s = lax.axis_index("s")
  wid = c * _NS + s

  _zero_flat_buf(zbuf, 224)
  for k in range(14):
    off = s * _PRPT + k * 224
    pltpu.sync_copy(zbuf, accd.at[pl.ds(off, 224)])
    pltpu.sync_copy(zbuf, accw.at[pl.ds(off, 224)])
  plsc.subcore_barrier()

  nsup = _PEPT // _PSCH

  def _sup(u, carry):
    off = wid * _PEPT + u * _PSCH
    pltpu.sync_copy(dst_hbm.at[pl.ds(off, _PSCH)], draw)
    pltpu.sync_copy(wa_hbm.at[pl.ds(off, _PSCH)], wraw)
    for jc in range(_PSCH // _CH):
      for j in range(8):
        o = jc * _CH + j * 16
        dv = draw[o:o + 16]
        ok = dv >= 0
        didx[jc, j * 16:(j + 1) * 16] = jnp.where(ok, dv, 0)
        oebuf[o:o + 16] = jnp.where(ok, 1.0, 0.0)
        wv = wraw[o:o + 16]
        webuf[o:o + 16] = jnp.where(ok, wv, 0.0)
    for jc in range(_PSCH // _CH):
      pltpu.sync_copy(oebuf.at[pl.ds(jc * _CH, _CH)],
                      accd.at[didx.at[jc]], add=True)
      pltpu.sync_copy(webuf.at[pl.ds(jc * _CH, _CH)],
                      accw.at[didx.at[jc]], add=True)
    return carry

  lax.fori_loop(0, nsup, _sup, None)
  plsc.subcore_barrier()

  off = s * _PRPT
  pltpu.sync_copy(accd.at[pl.ds(off, _PRPT)], cbuf)
  pltpu.sync_copy(cbuf, out_hbm.at[pl.ds((2 * c + 0) * _NP + off, _PRPT)])
  pltpu.sync_copy(accw.at[pl.ds(off, _PRPT)], cbuf)
  pltpu.sync_copy(cbuf, out_hbm.at[pl.ds((2 * c + 1) * _NP + off, _PRPT)])


_prep_call = functools.partial(
    pl.kernel,
    out_type=jax.ShapeDtypeStruct((4 * _NP,), jnp.float32),
    mesh=_mesh,
    scratch_types=[
        pltpu.VMEM((_PSCH,), jnp.int32),
        pltpu.VMEM((_PSCH,), jnp.float32),
        pltpu.VMEM((_PSCH // _CH, _CH), jnp.int32),
        pltpu.VMEM((_PSCH,), jnp.float32),
        pltpu.VMEM((_PSCH,), jnp.float32),
        pltpu.VMEM((224,), jnp.float32),
        pltpu.VMEM((_PRPT,), jnp.float32),
        pltpu.VMEM_SHARED((_NP,), jnp.float32),
        pltpu.VMEM_SHARED((_NP,), jnp.float32),
        pltpu.SemaphoreType.DMA,
    ],
)(_prep_body)


_CAPSEG = 25600
_ATOP = 25600


def _bucket_body(src_hbm, dst_hbm, wa_hbm, bsrc_hbm, bdst_hbm, bw_hbm,
                 cnt_hbm, sidx, draw, wraw, asrc, adst, aw, cbuf, sem):
  c = lax.axis_index("c")
  s = lax.axis_index("s")
  wid = c * _NS + s
  dump = _HALF + s
  lanev = lax.iota(jnp.int32, 16)

  def _sup(u, carry):
    off = wid * _PEPT + u * _PSCH
    pltpu.sync_copy(src_hbm.at[pl.ds(off, _PSCH)], sidx)
    pltpu.sync_copy(dst_hbm.at[pl.ds(off, _PSCH)], draw)
    pltpu.sync_copy(wa_hbm.at[pl.ds(off, _PSCH)], wraw)

    def _grp(g, cr):
      o0, o1 = cr
      o = g * 16
      sv = sidx[pl.ds(o, 16)]
      dv = draw[pl.ds(o, 16)]
      wv = wraw[pl.ds(o, 16)]
      m0 = (dv >= 0) & (dv < _HALF)
      m1 = dv >= _HALF
      cs0 = jnp.cumsum(jnp.where(m0, 1, 0))
      cs1 = jnp.cumsum(jnp.where(m1, 1, 0))
      pc0 = jnp.max(cs0)
      pc1 = jnp.max(cs1)
      pos0 = o0 + cs0 - 1
      plsc.store_scatter(asrc, [pos0], sv, mask=m0)
      plsc.store_scatter(adst, [pos0], dv, mask=m0)
      plsc.store_scatter(aw, [pos0], wv, mask=m0)
      pos1 = _ATOP - o1 - cs1
      plsc.store_scatter(asrc, [pos1], sv, mask=m1)
      plsc.store_scatter(adst, [pos1], dv - _HALF, mask=m1)
      plsc.store_scatter(aw, [pos1], wv, mask=m1)
      return (o0 + pc0, o1 + pc1)

    return lax.fori_loop(0, _PSCH // 16, _grp, carry)

  zero = jnp.int32(0)
  off0, off1 = lax.fori_loop(0, _PEPT // _PSCH, _sup, (zero, zero))

  zi = jnp.zeros((16,), jnp.int32)
  zf = jnp.zeros((16,), jnp.float32)
  dv16 = jnp.full((16,), dump, jnp.int32)

  def _pad0(i, o0):
    idx = o0 + lanev
    plsc.store_scatter(asrc, [idx], zi)
    plsc.store_scatter(adst, [idx], dv16)
    plsc.store_scatter(aw, [idx], zf)
    return o0 + 16

  npad0 = ((-off0) % 256 + 15) // 16
  _ = lax.fori_loop(0, npad0, _pad0, off0)
  off0r = off0 + ((-off0) % 256)

  def _pad1(i, o1):
    idx = _ATOP - o1 - 16 + lanev
    plsc.store_scatter(asrc, [idx], zi)
    plsc.store_scatter(adst, [idx], dv16)
    plsc.store_scatter(aw, [idx], zf)
    return o1 + 16

  npad1 = ((-off1) % 256 + 15) // 16
  _ = lax.fori_loop(0, npad1, _pad1, off1)
  off1r = off1 + ((-off1) % 256)
  plsc.subcore_barrier()

  seg0 = wid * _CAPSEG

  def _out0(i, carry):
    o = i * 128
    pltpu.sync_copy(asrc.at[pl.ds(o, 128)], bsrc_hbm.at[pl.ds(seg0 + o, 128)])
    pltpu.sync_copy(adst.at[pl.ds(o, 128)], bdst_hbm.at[pl.ds(seg0 + o, 128)])
    pltpu.sync_copy(aw.at[pl.ds(o, 128)], bw_hbm.at[pl.ds(seg0 + o, 128)])
    return carry

  lax.fori_loop(0, off0r // 128, _out0, None)

  seg1 = (32 + wid) * _CAPSEG
  start1 = pl.multiple_of(_ATOP - off1r, 128)

  def _out1(i, carry):
    o = i * 128
    pltpu.sync_copy(asrc.at[pl.ds(start1 + o, 128)],
                    bsrc_hbm.at[pl.ds(seg1 + o, 128)])
    pltpu.sync_copy(adst.at[pl.ds(start1 + o, 128)],
                    bdst_hbm.at[pl.ds(seg1 + o, 128)])
    pltpu.sync_copy(aw.at[pl.ds(start1 + o, 128)],
                    bw_hbm.at[pl.ds(seg1 + o, 128)])
    return carry

  lax.fori_loop(0, off1r // 128, _out1, None)

  cbuf[pl.ds(0, 16)] = jnp.where(lanev < 8, off0r, off1r)
  pltpu.sync_copy(cbuf.at[pl.ds(0, 8)], cnt_hbm.at[pl.ds(wid * 8, 8)])
  pltpu.sync_copy(cbuf.at[pl.ds(8, 8)],
                  cnt_hbm.at[pl.ds((32 + wid) * 8, 8)])


_bucket_call = functools.partial(
    pl.kernel,
    compiler_params=pltpu.CompilerParams(needs_layout_passes=False),
    out_type=(
        jax.ShapeDtypeStruct((2 * 32 * _CAPSEG,), jnp.int32),
        jax.ShapeDtypeStruct((2 * 32 * _CAPSEG,), jnp.int32),
        jax.ShapeDtypeStruct((2 * 32 * _CAPSEG,), jnp.float32),
        jax.ShapeDtypeStruct((2 * 32 * 8,), jnp.int32),
    ),
    mesh=_mesh,
    scratch_types=[
        pltpu.VMEM((_PSCH,), jnp.int32),
        pltpu.VMEM((_PSCH,), jnp.int32),
        pltpu.VMEM((_PSCH,), jnp.float32),
        pltpu.VMEM((_ATOP + 16,), jnp.int32),
        pltpu.VMEM((_ATOP + 16,), jnp.int32),
        pltpu.VMEM((_ATOP + 16,), jnp.float32),
        pltpu.VMEM((16,), jnp.int32),
        pltpu.SemaphoreType.DMA,
    ],
)(_bucket_body)


def _edge_body(has_w, a_hbm, bsrc_hbm, bdst_hbm, bw_hbm, cnt_hbm, out_hbm,
               cbuf2, si0, si1, d20, d21, wb0, wb1, rows0, rows1, acc,
               gs0, gs1, ss0, ss1):
  c = lax.axis_index("c")
  s = lax.axis_index("s")
  base = c * _HALF

  _zero_vec_buf(rows0, 112, _D)
  for k in range(14):
    pltpu.sync_copy(rows0.at[pl.ds(0, 112)],
                    acc.at[pl.ds(s * _RPT + k * 112, 112)])
  plsc.subcore_barrier()

  pltpu.sync_copy(cnt_hbm.at[pl.ds((c * 32 + 2 * s) * 8, 16)], cbuf2)
  v = cbuf2[pl.ds(0, 16)]
  lanev = lax.iota(jnp.int32, 16)
  cnt0 = jnp.max(jnp.where(lanev < 8, v, 0))
  cnt1 = jnp.max(jnp.where(lanev >= 8, v, 0))

  sets = ((si0, d20, wb0, rows0, gs0, ss0), (si1, d21, wb1, rows1, gs1, ss1))

  for seg in range(2):
    t = 2 * s + seg
    segoff = (c * 32 + t) * _CAPSEG
    npair = (cnt0 if seg == 0 else cnt1) // 256

    def _pair(i, carry, segoff=segoff, npair=npair):
      @pl.when(i > 0)
      def _drain_prev():
        pltpu.make_async_copy(rows0, acc.at[d20.at[0]], ss0).wait()
        pltpu.make_async_copy(rows1, acc.at[d21.at[0]], ss1).wait()

      gd = [None, None]
      for k in range(2):
        si_k, d2_k, wb_k, rows_k, gs_k, _ = sets[k]
        choff = segoff + i * 256 + k * 128
        pltpu.sync_copy(bsrc_hbm.at[pl.ds(choff, 128)], si_k)
        pltpu.sync_copy(bdst_hbm.at[pl.ds(choff, 128)], d2_k.at[0])
        if has_w:
          pltpu.sync_copy(bw_hbm.at[pl.ds(choff, 128)], wb_k)
        gd[k] = pltpu.async_copy(a_hbm.at[si_k], rows_k, gs_k)
      for k in range(2):
        si_k, d2_k, wb_k, rows_k, _, ss_k = sets[k]
        gd[k].wait()
        if has_w:
          def _scale(e, cc, wb=wb_k, rows=rows_k):
            g = (e // 16) * 16
            w16 = wb[pl.ds(g, 16)]
            wsp = w16.at[jnp.full((16,), e - g, jnp.int32)].get(
                mode="promise_in_bounds")
            for q in range(4):
              sl = slice(q * 16, (q + 1) * 16)
              rows[e, sl] = rows[e, sl] * wsp
            return cc

          lax.fori_loop(0, _CH, _scale, None)
        pltpu.async_copy(rows_k, acc.at[d2_k.at[0]], ss_k, add=True)
      return carry

    lax.fori_loop(0, npair, _pair, None)

    @pl.when(npair > 0)
    def _drain_tail():
      pltpu.make_async_copy(rows0, acc.at[d20.at[0]], ss0).wait()
      pltpu.make_async_copy(rows1, acc.at[d21.at[0]], ss1).wait()

  plsc.subcore_barrier()

  for k in range(7):
    off = s * _RPT + k * 224
    pltpu.sync_copy(acc.at[pl.ds(off, 224)],
                    out_hbm.at[pl.ds(base + off, 224)])


def _make_edge_call(has_w):
  return functools.partial(
      pl.kernel,
      out_type=jax.ShapeDtypeStruct((_NP, _D), jnp.float32),
      mesh=_mesh,
      compiler_params=pltpu.CompilerParams(use_tc_tiling_on_sc=False,
                                           needs_layout_passes=False),
      scratch_types=[
          pltpu.VMEM((16,), jnp.int32),
          pltpu.VMEM((_CH,), jnp.int32),
          pltpu.VMEM((_CH,), jnp.int32),
          pltpu.VMEM((1, _CH), jnp.int32),
          pltpu.VMEM((1, _CH), jnp.int32),
          pltpu.VMEM((_CH,), jnp.float32),
          pltpu.VMEM((_CH,), jnp.float32),
          pltpu.VMEM((_CH, _D), jnp.float32),
          pltpu.VMEM((_CH, _D), jnp.float32),
          pltpu.VMEM_SHARED((_HALF + _NS, _D), jnp.float32),
          pltpu.SemaphoreType.DMA,
          pltpu.SemaphoreType.DMA,
          pltpu.SemaphoreType.DMA,
          pltpu.SemaphoreType.DMA,
      ],
  )(functools.partial(_edge_body, has_w))


_edge_call_w = _make_edge_call(True)
_edge_call_nw = _make_edge_call(False)


def _deg_from_ref(dw_ref, col):
  dw = jnp.reshape(dw_ref[...], (4, _BLK))
  return (dw[0 + col] + dw[2 + col])[:, None]


def _pre_math(h, degv, w1, b1, w2, w3, b3):
  a = jnp.dot(h, w1, preferred_element_type=jnp.float32) + b1
  b = jnp.dot(h, w2, preferred_element_type=jnp.float32)
  base = (jnp.dot(h, w3, preferred_element_type=jnp.float32) + b3
          - degv * b)
  return a, base


def _tc_pre_x_body(col, x_ref, dw_ref, w1_ref, b1_ref, w2_ref, w3_ref,
                   b3_ref, a_ref, base_ref):
  degv = _deg_from_ref(dw_ref, col)
  a, base = _pre_math(x_ref[...], degv, w1_ref[...], b1_ref[...],
                      w2_ref[...], w3_ref[...], b3_ref[...])
  a_ref[...] = a
  base_ref[...] = base


def _tc_combine_pre_body(col, relu, bprev_ref, s_ref, dw_ref, w1_ref, b1_ref,
                         w2_ref, w3_ref, b3_ref, a_ref, base_ref):
  h = bprev_ref[...] + s_ref[...]
  if relu:
    h = jnp.maximum(h, 0.0)
  degv = _deg_from_ref(dw_ref, col)
  a, base = _pre_math(h, degv, w1_ref[...], b1_ref[...], w2_ref[...],
                      w3_ref[...], b3_ref[...])
  a_ref[...] = a
  base_ref[...] = base


def _readout_update(h, bo, rmax_ref, rsum_ref, rcnt_ref):
  i = pl.program_id(0)

  @pl.when(i == 0)
  def _init():
    rmax_ref[...] = jnp.full((_G, _D), -jnp.inf, jnp.float32)
    rsum_ref[...] = jnp.zeros((_G, _D), jnp.float32)
    rcnt_ref[...] = jnp.zeros((_G, _D), jnp.float32)

  dn = (((0,), (0,)), ((), ()))
  rsum_ref[...] += lax.dot_general(bo, h, dn,
                                   preferred_element_type=jnp.float32)
  rcnt_ref[...] += lax.dot_general(bo, jnp.ones_like(h), dn,
                                   preferred_element_type=jnp.float32)
  parts = [
      jnp.max(jnp.where(bo[:, g:g + 1] > 0.5, h, -jnp.inf), axis=0,
              keepdims=True)
      for g in range(_G)
  ]
  rmax_ref[...] = jnp.maximum(rmax_ref[...], jnp.concatenate(parts, axis=0))


def _tc_combine_read_pre_body(col, bprev_ref, s_ref, dw_ref, bo_ref,
                              w1_ref, b1_ref, w2_ref, w3_ref, b3_ref,
                              a_ref, base_ref, rmax_ref, rsum_ref, rcnt_ref):
  h = bprev_ref[...] + s_ref[...]
  _readout_update(h, bo_ref[...], rmax_ref, rsum_ref, rcnt_ref)
  degv = _deg_from_ref(dw_ref, col)
  a, base = _pre_math(h, degv, w1_ref[...], b1_ref[...], w2_ref[...],
                      w3_ref[...], b3_ref[...])
  a_ref[...] = a
  base_ref[...] = base


def _tc_final_body(bprev_ref, s_ref, bo_ref, m1_ref, s1_ref, c1_ref,
                   l1w_ref, l1b_ref, l2w_ref, l2b_ref, out_ref,
                   rmax_ref, rsum_ref, rcnt_ref):
  h = bprev_ref[...] + s_ref[...]
  _readout_update(h, bo_ref[...], rmax_ref, rsum_ref, rcnt_ref)

  @pl.when(pl.program_id(0) == _NB - 1)
  def _fin():
    mean1 = s1_ref[...] / jnp.maximum(c1_ref[...], 1.0)
    mean2 = rsum_ref[...] / jnp.maximum(rcnt_ref[...], 1.0)
    z = (jnp.concatenate([m1_ref[...], mean1], axis=1)
         + jnp.concatenate([rmax_ref[...], mean2], axis=1))
    y = jnp.maximum(
        jnp.dot(z, l1w_ref[...], preferred_element_type=jnp.float32)
        + l1b_ref[...], 0.0)
    out_ref[...] = (jnp.sum(y * l2w_ref[...], axis=1, keepdims=True)
                    + l2b_ref[0, 0])


_blk_spec = pl.BlockSpec((_BLK, _D), lambda i: (i, 0))
_dw_spec = pl.BlockSpec((4, 1, 1, _BLK), lambda i: (0, i, 0, 0))
_bo_spec = pl.BlockSpec((_BLK, _G), lambda i: (i, 0))


def _full_spec(shape):
  nd = len(shape)
  return pl.BlockSpec(shape, lambda i: (0,) * nd)


_w_spec = _full_spec((_D, _D))
_b_spec = _full_spec((1, _D))
_g_spec = _full_spec((_G, _D))

_nd_out = jax.ShapeDtypeStruct((_NP, _D), jnp.float32)
_g_out = jax.ShapeDtypeStruct((_G, _D), jnp.float32)


def _tc_pre_x(xp, dw4, p):
  return pl.pallas_call(
      functools.partial(_tc_pre_x_body, 0),
      grid=(_NB,),
      in_specs=[_blk_spec, _dw_spec, _w_spec, _b_spec, _w_spec, _w_spec,
                _b_spec],
      out_specs=[_blk_spec, _blk_spec],
      out_shape=[_nd_out, _nd_out],
  )(xp, dw4, p["W1"], p["b1"].reshape(1, _D), p["W2"], p["W3"],
    p["b3"].reshape(1, _D))


def _tc_combine_pre(col, relu, bprev, sarr, dw4, p):
  return pl.pallas_call(
      functools.partial(_tc_combine_pre_body, col, relu),
      grid=(_NB,),
      in_specs=[_blk_spec, _blk_spec, _dw_spec, _w_spec, _b_spec, _w_spec,
                _w_spec, _b_spec],
      out_specs=[_blk_spec, _blk_spec],
      out_shape=[_nd_out, _nd_out],
  )(bprev, sarr, dw4, p["W1"], p["b1"].reshape(1, _D), p["W2"], p["W3"],
    p["b3"].reshape(1, _D))


def _tc_combine_read_pre(col, bprev, sarr, dw4, bo, p):
  return pl.pallas_call(
      functools.partial(_tc_combine_read_pre_body, col),
      grid=(_NB,),
      in_specs=[_blk_spec, _blk_spec, _dw_spec, _bo_spec, _w_spec,
                _b_spec, _w_spec, _w_spec, _b_spec],
      out_specs=[_blk_spec, _blk_spec, _g_spec, _g_spec, _g_spec],
      out_shape=[_nd_out, _nd_out, _g_out, _g_out, _g_out],
  )(bprev, sarr, dw4, bo, p["W1"], p["b1"].reshape(1, _D), p["W2"],
    p["W3"], p["b3"].reshape(1, _D))


def _tc_final(bprev, sarr, bo, m1, s1, c1, params):
  return pl.pallas_call(
      _tc_final_body,
      grid=(_NB,),
      in_specs=[_blk_spec, _blk_spec, _bo_spec, _g_spec, _g_spec, _g_spec,
                _full_spec((2 * _D, _D)), _b_spec, _b_spec,
                _full_spec((1, 1))],
      out_specs=pl.BlockSpec((_G, 1), lambda i: (0, 0)),
      out_shape=jax.ShapeDtypeStruct((_G, 1), jnp.float32),
      scratch_shapes=[
          pltpu.VMEM((_G, _D), jnp.float32),
          pltpu.VMEM((_G, _D), jnp.float32),
          pltpu.VMEM((_G, _D), jnp.float32),
      ],
  )(bprev, sarr, bo, m1, s1, c1, params["lin1_W"],
    params["lin1_b"].reshape(1, _D), params["lin2_W"].reshape(1, _D),
    params["lin2_b"].reshape(1, 1))


def kernel(x, edge_index, edge_attr, batch, params):
  src = edge_index[0].astype(jnp.int32)
  dst = edge_index[1].astype(jnp.int32)
  pad_e = _EP - _E
  src_p = jnp.concatenate([src, jnp.zeros((pad_e,), jnp.int32)])
  dst_p = jnp.concatenate([dst, jnp.full((pad_e,), -1, jnp.int32)])
  w_attr = jnp.concatenate([edge_attr.astype(jnp.float32),
                            jnp.zeros((pad_e,), jnp.float32)])
  xp = jnp.pad(x, ((0, _NP - _N), (0, 0)))
  batch_p = jnp.concatenate(
      [batch.astype(jnp.int32), jnp.full((_NP - _N,), 127, jnp.int32)])
  bo = (batch_p[:, None] == jnp.arange(_G, dtype=jnp.int32)[None, :]
        ).astype(jnp.float32)

  dw4 = _prep_call(dst_p, w_attr).reshape(4, _NB, 1, _BLK)
  bsrc, bdst, bw, cnts = _bucket_call(src_p, dst_p, w_attr)

  a1, base1 = _tc_pre_x(xp, dw4, params["conv1"])
  s1 = _edge_call_nw(a1, bsrc, bdst, bw, cnts)
  a2, base2 = _tc_combine_pre(1, True, base1, s1, dw4, params["pool1"])
  s2 = _edge_call_w(a2, bsrc, bdst, bw, cnts)
  a3, base3, m1, su1, c1 = _tc_combine_read_pre(
      0, base2, s2, dw4, bo, params["conv2"])
  s3 = _edge_call_nw(a3, bsrc, bdst, bw, cnts)
  a4, base4 = _tc_combine_pre(1, True, base3, s3, dw4, params["pool2"])
  s4 = _edge_call_w(a4, bsrc, bdst, bw, cnts)
  return _tc_final(base4, s4, bo, m1, su1, c1, params)
